# baseline (speedup 1.0000x reference)
import functools

import jax
import jax.numpy as jnp
from jax import lax
from jax.experimental import pallas as pl
from jax.experimental.pallas import tpu as pltpu


def _fused_kernel(x_ref, w_ref, out_ref, lscr, sscr, *, k, blk, split, bpb):
    i = pl.program_id(0)

    @pl.when(i < split)
    def _logits():
        lg = lax.dot_general(
            w_ref[...], x_ref[...], (((1,), (1,)), ((), ())),
            preferred_element_type=jnp.float32)
        lscr[i // bpb, pl.ds((i % bpb) * blk, blk)] = lg.reshape(blk)

    @pl.when(i == split)
    def _route():
        l = lscr[...]
        nb, ns = l.shape
        u = lax.bitcast_convert_type(l, jnp.int32)
        key = u ^ (jnp.int32(0x7FFFFFFF) & (u >> 31))

        lo = jnp.min(key, axis=1, keepdims=True)
        hi = jnp.max(key, axis=1, keepdims=True)

        def body(_, lh):
            lo, hi = lh
            xo = lo ^ hi
            mid = (lo & hi) + (xo >> 1) + (xo & 1)
            cnt = jnp.sum((key >= mid).astype(jnp.int32), axis=1,
                          keepdims=True)
            ge = cnt >= k
            return jnp.where(ge, mid, lo), jnp.where(ge, hi, mid - 1)

        lo, hi = lax.fori_loop(0, 34, body, (lo, hi))
        t = lo

        gt = key > t
        eq = key == t
        cnt_gt = jnp.sum(gt.astype(jnp.int32), axis=1, keepdims=True)
        r = k - cnt_gt
        iota = lax.broadcasted_iota(jnp.int32, (nb, ns), 1)

        lo2 = jnp.ones_like(r)
        hi2 = jnp.full_like(r, ns)

        def body2(_, lh):
            lo, hi = lh
            mid = (lo + hi) >> 1
            cnt = jnp.sum((eq & (iota < mid)).astype(jnp.int32), axis=1,
                          keepdims=True)
            ge = cnt >= r
            return jnp.where(ge, lo, mid + 1), jnp.where(ge, mid, hi)

        lo2, _ = lax.fori_loop(0, 14, body2, (lo2, hi2))
        selected = gt | (eq & (iota < lo2))

        m = jnp.max(l, axis=1, keepdims=True)
        e = jnp.exp(l - m)
        denom = jnp.sum(jnp.where(selected, e, 0.0), axis=1, keepdims=True)
        sscr[...] = 1.0 + jnp.where(selected, e / denom, 0.0)

    @pl.when(i >= split)
    def _apply():
        j = i - split
        s_col = jnp.reshape(sscr[j // bpb, pl.ds((j % bpb) * blk, blk)],
                            (blk, 1))
        out_ref[...] = x_ref[...] * s_col


def kernel(x, W_router):
    b, s, d = x.shape
    k = int(s * 0.5)
    blk = 2048
    bs = b * s
    split = bs // blk
    bpb = s // blk
    xf = x.reshape(bs, d)
    wt = W_router.reshape(1, d)

    out = pl.pallas_call(
        functools.partial(_fused_kernel, k=k, blk=blk, split=split, bpb=bpb),
        grid=(2 * split,),
        in_specs=[
            pl.BlockSpec((blk, d), lambda i, split=split: (i % split, 0)),
            pl.BlockSpec((1, d), lambda i: (0, 0)),
        ],
        out_specs=pl.BlockSpec(
            (blk, d), lambda i, split=split: (jnp.maximum(i - split, 0), 0)),
        out_shape=jax.ShapeDtypeStruct((bs, d), jnp.float32),
        scratch_shapes=[
            pltpu.VMEM((b, s), jnp.float32),
            pltpu.VMEM((b, s), jnp.float32),
        ],
    )(xf, wt)

    return out.reshape(b, s, d)


# speedup vs baseline: 1.2426x; 1.2426x over previous
import functools

import jax
import jax.numpy as jnp
from jax import lax
from jax.experimental import pallas as pl
from jax.experimental.pallas import tpu as pltpu


def _fused_kernel(x_ref, w_ref, out_ref, lscr, sscr, cache, *, k, blk, split,
                  bpb):
    i = pl.program_id(0)

    @pl.when(i < split)
    def _logits():
        xb = x_ref[...]
        lg = lax.dot_general(
            w_ref[...], xb, (((1,), (1,)), ((), ())),
            preferred_element_type=jnp.float32)
        lscr[i // bpb, pl.ds((i % bpb) * blk, blk)] = lg.reshape(blk)
        cache[pl.ds(i * blk, blk), :] = xb.astype(jnp.bfloat16)

    @pl.when(i == split)
    def _route():
        l = lscr[...]
        nb, ns = l.shape
        u = lax.bitcast_convert_type(l, jnp.int32)
        key = u ^ (jnp.int32(0x7FFFFFFF) & (u >> 31))

        lo = jnp.min(key, axis=1, keepdims=True)
        hi = jnp.max(key, axis=1, keepdims=True)

        def body(_, lh):
            lo, hi = lh
            xo = lo ^ hi
            mid = (lo & hi) + (xo >> 1) + (xo & 1)
            cnt = jnp.sum((key >= mid).astype(jnp.int32), axis=1,
                          keepdims=True)
            ge = cnt >= k
            return jnp.where(ge, mid, lo), jnp.where(ge, hi, mid - 1)

        lo, hi = lax.fori_loop(0, 34, body, (lo, hi))
        t = lo

        gt = key > t
        eq = key == t
        cnt_gt = jnp.sum(gt.astype(jnp.int32), axis=1, keepdims=True)
        r = k - cnt_gt
        iota = lax.broadcasted_iota(jnp.int32, (nb, ns), 1)

        lo2 = jnp.ones_like(r)
        hi2 = jnp.full_like(r, ns)

        def body2(_, lh):
            lo, hi = lh
            mid = (lo + hi) >> 1
            cnt = jnp.sum((eq & (iota < mid)).astype(jnp.int32), axis=1,
                          keepdims=True)
            ge = cnt >= r
            return jnp.where(ge, lo, mid + 1), jnp.where(ge, mid, hi)

        lo2, _ = lax.fori_loop(0, 14, body2, (lo2, hi2))
        selected = gt | (eq & (iota < lo2))

        m = jnp.max(l, axis=1, keepdims=True)
        e = jnp.exp(l - m)
        denom = jnp.sum(jnp.where(selected, e, 0.0), axis=1, keepdims=True)
        sscr[...] = 1.0 + jnp.where(selected, e / denom, 0.0)

    @pl.when(i >= split)
    def _apply():
        j = i - split
        s_col = jnp.reshape(sscr[j // bpb, pl.ds((j % bpb) * blk, blk)],
                            (blk, 1))
        xb = cache[pl.ds(j * blk, blk), :].astype(jnp.float32)
        out_ref[...] = xb * s_col


def kernel(x, W_router):
    b, s, d = x.shape
    k = int(s * 0.5)
    blk = 1024
    bs = b * s
    split = bs // blk
    bpb = s // blk
    xf = x.reshape(bs, d)
    wt = W_router.reshape(1, d)

    out = pl.pallas_call(
        functools.partial(_fused_kernel, k=k, blk=blk, split=split, bpb=bpb),
        grid=(2 * split,),
        in_specs=[
            pl.BlockSpec(
                (blk, d),
                lambda i, split=split: (jnp.minimum(i, split - 1), 0)),
            pl.BlockSpec((1, d), lambda i: (0, 0)),
        ],
        out_specs=pl.BlockSpec(
            (blk, d), lambda i, split=split: (jnp.maximum(i - split, 0), 0)),
        out_shape=jax.ShapeDtypeStruct((bs, d), jnp.float32),
        scratch_shapes=[
            pltpu.VMEM((b, s), jnp.float32),
            pltpu.VMEM((b, s), jnp.float32),
            pltpu.VMEM((bs, d), jnp.bfloat16),
        ],
        compiler_params=pltpu.CompilerParams(
            vmem_limit_bytes=120 * 1024 * 1024),
    )(xf, wt)

    return out.reshape(b, s, d)


# speedup vs baseline: 1.2602x; 1.0141x over previous
import functools

import jax
import jax.numpy as jnp
from jax import lax
from jax.experimental import pallas as pl
from jax.experimental.pallas import tpu as pltpu


def _fused_kernel(x_ref, w_ref, out_ref, lscr, sscr, cache, *, k, blk, split,
                  bpb):
    i = pl.program_id(0)

    @pl.when(i < split)
    def _logits():
        xb = x_ref[...]
        lg = lax.dot_general(
            w_ref[...], xb, (((1,), (1,)), ((), ())),
            preferred_element_type=jnp.float32)
        lscr[i // bpb, pl.ds((i % bpb) * blk, blk)] = lg.reshape(blk)
        cache[pl.ds(i * blk, blk), :] = xb.astype(jnp.bfloat16)

    @pl.when(i == split)
    def _route():
        l = lscr[...]
        nb, ns = l.shape
        u = lax.bitcast_convert_type(l, jnp.int32)
        key = u ^ (jnp.int32(0x7FFFFFFF) & (u >> 31))

        lo = jnp.min(key, axis=1, keepdims=True)
        hi = jnp.max(key, axis=1, keepdims=True)

        def srch_cond(lh):
            lo, hi = lh
            return jnp.any(lo < hi)

        def body(lh):
            lo, hi = lh
            xo = lo ^ hi
            mid = (lo & hi) + (xo >> 1) + (xo & 1)
            cnt = jnp.sum((key >= mid).astype(jnp.int32), axis=1,
                          keepdims=True)
            ge = cnt >= k
            return jnp.where(ge, mid, lo), jnp.where(ge, hi, mid - 1)

        lo, hi = lax.while_loop(srch_cond, body, (lo, hi))
        t = lo

        gt = key > t
        eq = key == t
        cnt_gt = jnp.sum(gt.astype(jnp.int32), axis=1, keepdims=True)
        cnt_eq = jnp.sum(eq.astype(jnp.int32), axis=1, keepdims=True)
        r = k - cnt_gt
        iota = lax.broadcasted_iota(jnp.int32, (nb, ns), 1)

        def tie_search(_):
            lo2 = jnp.ones_like(r)
            hi2 = jnp.full_like(r, ns)

            def body2(_, lh):
                lo, hi = lh
                mid = (lo + hi) >> 1
                cnt = jnp.sum((eq & (iota < mid)).astype(jnp.int32), axis=1,
                              keepdims=True)
                ge = cnt >= r
                return jnp.where(ge, lo, mid + 1), jnp.where(ge, mid, hi)

            return lax.fori_loop(0, 14, body2, (lo2, hi2))[0]

        lo2 = lax.cond(jnp.any(cnt_eq > r), tie_search,
                       lambda _: jnp.full_like(r, ns), None)
        selected = gt | (eq & (iota < lo2))

        m = jnp.max(l, axis=1, keepdims=True)
        e = jnp.exp(l - m)
        denom = jnp.sum(jnp.where(selected, e, 0.0), axis=1, keepdims=True)
        sscr[...] = 1.0 + jnp.where(selected, e / denom, 0.0)

    @pl.when(i >= split)
    def _apply():
        j = i - split
        s_col = jnp.reshape(sscr[j // bpb, pl.ds((j % bpb) * blk, blk)],
                            (blk, 1))
        xb = cache[pl.ds(j * blk, blk), :].astype(jnp.float32)
        out_ref[...] = xb * s_col


def kernel(x, W_router):
    b, s, d = x.shape
    k = int(s * 0.5)
    blk = 1024
    bs = b * s
    split = bs // blk
    bpb = s // blk
    xf = x.reshape(bs, d)
    wt = W_router.reshape(1, d)

    out = pl.pallas_call(
        functools.partial(_fused_kernel, k=k, blk=blk, split=split, bpb=bpb),
        grid=(2 * split,),
        in_specs=[
            pl.BlockSpec(
                (blk, d),
                lambda i, split=split: (jnp.minimum(i, split - 1), 0)),
            pl.BlockSpec((1, d), lambda i: (0, 0)),
        ],
        out_specs=pl.BlockSpec(
            (blk, d), lambda i, split=split: (jnp.maximum(i - split, 0), 0)),
        out_shape=jax.ShapeDtypeStruct((bs, d), jnp.float32),
        scratch_shapes=[
            pltpu.VMEM((b, s), jnp.float32),
            pltpu.VMEM((b, s), jnp.float32),
            pltpu.VMEM((bs, d), jnp.bfloat16),
        ],
        compiler_params=pltpu.CompilerParams(
            vmem_limit_bytes=120 * 1024 * 1024),
    )(xf, wt)

    return out.reshape(b, s, d)
